# baseline (device time: 13606 ns/iter reference)
import jax
import jax.numpy as jnp
from jax import lax
from jax.experimental import pallas as pl
from jax.experimental.pallas import tpu as pltpu

N_DEV = 4

_BLOCK_ORDER = (2, 1, 3, 0)
_WAIT_ORDER = (1, 3, 2)


def kernel(x, w_mat):
    m, k = x.shape
    n = w_mat.shape[1]
    nblk = n // N_DEV

    def body(
        x_hbm, w_hbm, out_hbm,
        x_vmem, w_vmem, stage_ref, send_ref, obuf_ref,
        ld_sems, out_sems, send_sems, recv_sems,
    ):
        my = lax.axis_index("i")

        ld_x = pltpu.make_async_copy(x_hbm, x_vmem, ld_sems.at[0])
        ld_x.start()
        ld_w = {}
        for idx, d in enumerate(_BLOCK_ORDER):
            tgt = (my + d) % N_DEV
            cp = pltpu.make_async_copy(
                w_hbm.at[:, pl.ds(tgt * nblk, nblk)],
                w_vmem.at[idx],
                ld_sems.at[idx + 1],
            )
            cp.start()
            ld_w[d] = cp

        barrier = pltpu.get_barrier_semaphore()
        for d in (1, 2, 3):
            pl.semaphore_signal(
                barrier, inc=1,
                device_id=((my + d) % N_DEV,),
                device_id_type=pl.DeviceIdType.MESH,
            )
        pl.semaphore_wait(barrier, N_DEV - 1)

        ld_x.wait()
        x_bf = x_vmem[:, :].astype(jnp.bfloat16)

        rdmas = {}
        out_dmas = []
        for idx, d in enumerate(_BLOCK_ORDER):
            tgt = (my + d) % N_DEV
            ld_w[d].wait()
            w_bf = w_vmem[idx].astype(jnp.bfloat16)
            blk = jnp.dot(x_bf, w_bf, preferred_element_type=jnp.float32)
            blk = jnp.maximum(blk, 0.0)
            if d == 0:
                obuf_ref[pl.ds(my * m, m), :] = blk
                dma = pltpu.make_async_copy(
                    obuf_ref.at[pl.ds(my * m, m), :],
                    out_hbm.at[pl.ds(my * m, m), :],
                    out_sems.at[3],
                )
                dma.start()
                out_dmas.append(dma)
            else:
                send_ref[d - 1] = blk.astype(jnp.bfloat16)
                rdma = pltpu.make_async_remote_copy(
                    src_ref=send_ref.at[d - 1],
                    dst_ref=stage_ref.at[pl.ds(my * m, m), :],
                    send_sem=send_sems.at[d - 1],
                    recv_sem=recv_sems.at[d - 1],
                    device_id=(tgt,),
                    device_id_type=pl.DeviceIdType.MESH,
                )
                rdma.start()
                rdmas[d] = rdma

        for d in _WAIT_ORDER:
            rdmas[d].wait_recv()
            src = (my - d) % N_DEV
            obuf_ref[pl.ds(src * m, m), :] = stage_ref[
                pl.ds(src * m, m), :
            ].astype(jnp.float32)
            dma = pltpu.make_async_copy(
                obuf_ref.at[pl.ds(src * m, m), :],
                out_hbm.at[pl.ds(src * m, m), :],
                out_sems.at[d - 1],
            )
            dma.start()
            out_dmas.append(dma)

        for dma in out_dmas:
            dma.wait()
        for d in _WAIT_ORDER:
            rdmas[d].wait_send()

    return pl.pallas_call(
        body,
        out_shape=jax.ShapeDtypeStruct((n, nblk), jnp.float32),
        in_specs=[
            pl.BlockSpec(memory_space=pl.ANY),
            pl.BlockSpec(memory_space=pl.ANY),
        ],
        out_specs=pl.BlockSpec(memory_space=pl.ANY),
        scratch_shapes=[
            pltpu.VMEM((m, k), jnp.float32),
            pltpu.VMEM((N_DEV, k, nblk), jnp.float32),
            pltpu.VMEM((n, nblk), jnp.bfloat16),
            pltpu.VMEM((N_DEV - 1, m, nblk), jnp.bfloat16),
            pltpu.VMEM((n, nblk), jnp.float32),
            pltpu.SemaphoreType.DMA((N_DEV + 1,)),
            pltpu.SemaphoreType.DMA((N_DEV,)),
            pltpu.SemaphoreType.DMA((N_DEV - 1,)),
            pltpu.SemaphoreType.DMA((N_DEV - 1,)),
        ],
        compiler_params=pltpu.CompilerParams(collective_id=0),
    )(x, w_mat)


# device time: 12720 ns/iter; 1.0697x vs baseline; 1.0697x over previous
import jax
import jax.numpy as jnp
from jax import lax
from jax.experimental import pallas as pl
from jax.experimental.pallas import tpu as pltpu

N_DEV = 4

_BLOCK_ORDER = (2, 1, 3, 0)
_WAIT_ORDER = (1, 3, 2)


def kernel(x, w_mat):
    m, k = x.shape
    n = w_mat.shape[1]
    nblk = n // N_DEV

    def body(
        x_hbm, w_hbm, out_hbm,
        x_vmem, w_vmem, stage_ref, send_ref, obuf_ref,
        ld_sems, out_sems, send_sems, recv_sems,
    ):
        my = lax.axis_index("i")

        ld_x = pltpu.make_async_copy(x_hbm, x_vmem, ld_sems.at[0])
        ld_x.start()
        ld_w = {}
        for idx, d in enumerate(_BLOCK_ORDER):
            tgt = (my + d) % N_DEV
            cp = pltpu.make_async_copy(
                w_hbm.at[:, pl.ds(tgt * nblk, nblk)],
                w_vmem.at[idx],
                ld_sems.at[idx + 1],
            )
            cp.start()
            ld_w[d] = cp

        barrier = pltpu.get_barrier_semaphore()
        for d in (1, 2, 3):
            pl.semaphore_signal(
                barrier, inc=1,
                device_id=((my + d) % N_DEV,),
                device_id_type=pl.DeviceIdType.MESH,
            )
        pl.semaphore_wait(barrier, N_DEV - 1)

        ld_x.wait()
        x_bf = x_vmem[:, :].astype(jnp.bfloat16)

        rdmas = {}
        out_dmas = []
        for idx, d in enumerate(_BLOCK_ORDER):
            tgt = (my + d) % N_DEV
            ld_w[d].wait()
            w_bf = w_vmem[idx].astype(jnp.bfloat16)
            blk = jnp.dot(x_bf, w_bf, preferred_element_type=jnp.float32)
            blk = jnp.maximum(blk, 0.0)
            if d == 0:
                obuf_ref[pl.ds(my * m, m), :] = blk
                dma = pltpu.make_async_copy(
                    obuf_ref.at[pl.ds(my * m, m), :],
                    out_hbm.at[pl.ds(my * m, m), :],
                    out_sems.at[3],
                )
                dma.start()
                out_dmas.append(dma)
            else:
                send_ref[d - 1] = blk.astype(jnp.bfloat16)
                rdma = pltpu.make_async_remote_copy(
                    src_ref=send_ref.at[d - 1],
                    dst_ref=stage_ref.at[pl.ds(my * m, m), :],
                    send_sem=send_sems.at[d - 1],
                    recv_sem=recv_sems.at[d - 1],
                    device_id=(tgt,),
                    device_id_type=pl.DeviceIdType.MESH,
                )
                rdma.start()
                rdmas[d] = rdma

        for d in _WAIT_ORDER:
            rdmas[d].wait_recv()
            src = (my - d) % N_DEV
            obuf_ref[pl.ds(src * m, m), :] = stage_ref[
                pl.ds(src * m, m), :
            ].astype(jnp.float32)
            dma = pltpu.make_async_copy(
                obuf_ref.at[pl.ds(src * m, m), :],
                out_hbm.at[pl.ds(src * m, m), :],
                out_sems.at[d - 1],
            )
            dma.start()
            out_dmas.append(dma)

        for dma in out_dmas:
            dma.wait()
        for d in _WAIT_ORDER:
            rdmas[d].wait_send()

    return pl.pallas_call(
        body,
        out_shape=jax.ShapeDtypeStruct((n, nblk), jnp.float32),
        in_specs=[
            pl.BlockSpec(memory_space=pl.ANY),
            pl.BlockSpec(memory_space=pl.ANY),
        ],
        out_specs=pl.BlockSpec(memory_space=pl.ANY),
        scratch_shapes=[
            pltpu.VMEM((m, k), jnp.float32),
            pltpu.VMEM((N_DEV, k, nblk), jnp.float32),
            pltpu.VMEM((n, nblk), jnp.bfloat16),
            pltpu.VMEM((N_DEV - 1, m, nblk), jnp.bfloat16),
            pltpu.VMEM((n, nblk), jnp.float32),
            pltpu.SemaphoreType.DMA((N_DEV + 1,)),
            pltpu.SemaphoreType.DMA((N_DEV,)),
            pltpu.SemaphoreType.DMA((N_DEV - 1,)),
            pltpu.SemaphoreType.DMA((N_DEV - 1,)),
        ],
        compiler_params=pltpu.CompilerParams(
            collective_id=0,
            vmem_limit_bytes=100 * 1024 * 1024,
        ),
    )(x, w_mat)
